# baseline (speedup 1.0000x reference)
import functools

import jax
import jax.numpy as jnp
from jax import lax
from jax.experimental import pallas as pl
from jax.experimental.pallas import tpu as pltpu
from jax.experimental.pallas import tpu_sc as plsc

_B, _DIN, _D, _E, _H, _C = 2048, 2048, 1024, 8, 2048, 1000
_T = 256
_NT = 24
_PCAP = _T * _NT
_BT = 256
_NBT = _B // _BT
_LN = 128

_bf16 = jnp.bfloat16
_f32 = jnp.float32


def _trunk_gate(x, Wt, bt, Wg_pad, bg_pad):
    def body(x_ref, wt_ref, bt_ref, wg_ref, bg_ref, f_ref, p_ref, rp_ref):
        xb = x_ref[...].astype(_bf16)
        feat = jnp.dot(xb, wt_ref[...].astype(_bf16), preferred_element_type=_f32)
        featb = jnp.maximum(feat + bt_ref[...], 0.0).astype(_bf16)
        f_ref[...] = featb
        gl = jnp.dot(featb, wg_ref[...].astype(_bf16),
                     preferred_element_type=_f32) + bg_ref[...]
        lane = lax.broadcasted_iota(jnp.int32, (_BT, _LN), 1)
        gl = jnp.where(lane < _E, gl, -jnp.inf)
        m = jnp.max(gl, axis=1, keepdims=True)
        ex = jnp.exp(gl - m)
        p = ex / jnp.sum(ex, axis=1, keepdims=True)
        p_ref[...] = p[:, :_E]
        m1 = jnp.max(p, axis=1, keepdims=True)
        e1 = jnp.min(jnp.where(p >= m1, lane, _LN), axis=1, keepdims=True)
        p2 = jnp.where(lane == e1, -1.0, p)
        m2 = jnp.max(p2, axis=1, keepdims=True)
        e2 = jnp.min(jnp.where(p2 >= m2, lane, _LN), axis=1, keepdims=True)
        s = m1 + m2
        rp_ref[...] = (jnp.where(lane == 0, e1.astype(_f32), 0.0)
                       + jnp.where(lane == 1, e2.astype(_f32), 0.0)
                       + jnp.where(lane == 2, m1 / s, 0.0)
                       + jnp.where(lane == 3, m2 / s, 0.0))

    return pl.pallas_call(
        body,
        grid=(_NBT,),
        in_specs=[
            pl.BlockSpec((_BT, _DIN), lambda i: (i, 0)),
            pl.BlockSpec((_DIN, _D), lambda i: (0, 0)),
            pl.BlockSpec((_D,), lambda i: (0,)),
            pl.BlockSpec((_D, _LN), lambda i: (0, 0)),
            pl.BlockSpec((_LN,), lambda i: (0,)),
        ],
        out_specs=[
            pl.BlockSpec((_BT, _D), lambda i: (i, 0)),
            pl.BlockSpec((_BT, _E), lambda i: (i, 0)),
            pl.BlockSpec((_BT, _LN), lambda i: (i, 0)),
        ],
        out_shape=[
            jax.ShapeDtypeStruct((_B, _D), _bf16),
            jax.ShapeDtypeStruct((_B, _E), _f32),
            jax.ShapeDtypeStruct((_B, _LN), _f32),
        ],
    )(x, Wt, bt, Wg_pad, bg_pad)


def _cumsum_axis0(x, n):
    iot = lax.broadcasted_iota(jnp.int32, x.shape, 0)
    k = 1
    while k < n:
        sh = pltpu.roll(x, k, 0)
        x = x + jnp.where(iot >= k, sh, 0.0)
        k *= 2
    return x


def _route_meta(rp):
    def body(rp_ref, pe_ref, po_ref, te_ref):
        rpv = rp_ref[...]
        lane = lax.broadcasted_iota(jnp.int32, (_B, _LN), 1)
        e1 = rpv[:, 0:1].astype(jnp.int32)
        e2 = rpv[:, 1:2].astype(jnp.int32)
        oh1 = (lane == e1).astype(_f32)
        oh2 = (lane == e2).astype(_f32)
        cum1 = _cumsum_axis0(oh1, _B)
        cum2 = _cumsum_axis0(oh2, _B)
        counts1 = cum1[_B - 1:_B, :]
        counts = counts1 + cum2[_B - 1:_B, :]
        padded = jnp.floor((counts + (_T - 1)) * (1.0 / _T)) * _T
        liot = lax.broadcasted_iota(jnp.int32, (1, _LN), 1)
        ends = padded
        k = 1
        while k < _E:
            ends = ends + jnp.where(liot >= k, pltpu.roll(ends, k, 1), 0.0)
            k *= 2
        offs = ends - padded
        pos_e = jnp.sum(oh1 * (offs + cum1), axis=1, keepdims=True) - 1.0
        pos_o = jnp.sum(oh2 * (offs + counts1 + cum2), axis=1, keepdims=True) - 1.0
        pe_ref[...] = pos_e.astype(jnp.int32)
        po_ref[...] = pos_o.astype(jnp.int32)
        tile_start = (liot * _T).astype(_f32)
        te = jnp.zeros((1, _LN), _f32)
        for e in range(_E):
            end_e = jnp.sum(jnp.where(liot == e, ends, 0.0), axis=1, keepdims=True)
            te = te + (tile_start >= end_e).astype(_f32)
        total = jnp.sum(jnp.where(liot == _E - 1, ends, 0.0), axis=1, keepdims=True)
        active = (tile_start < total).astype(jnp.int32)
        te_ref[0:1, :] = jnp.minimum(te, _E - 1).astype(jnp.int32)
        te_ref[1:2, :] = active

    return pl.pallas_call(
        body,
        grid=(1,),
        in_specs=[pl.BlockSpec((_B, _LN), lambda i: (0, 0))],
        out_specs=[
            pl.BlockSpec((_B, 1), lambda i: (0, 0)),
            pl.BlockSpec((_B, 1), lambda i: (0, 0)),
            pl.BlockSpec((2, _LN), lambda i: (0, 0)),
        ],
        out_shape=[
            jax.ShapeDtypeStruct((_B, 1), jnp.int32),
            jax.ShapeDtypeStruct((_B, 1), jnp.int32),
            jax.ShapeDtypeStruct((2, _LN), jnp.int32),
        ],
    )(rp)


def _mlp(featb, pos_e, pos_o, W1, b1, W2, b2, tile_expert):
    def body(te_ref, fb_ref, pe_ref, po_ref, w1_ref, b1_ref, w2_ref, b2_ref, out_ref):
        gidx = (lax.broadcasted_iota(jnp.int32, (_B, _T), 1)
                + pl.program_id(0) * _T)

        @pl.when(te_ref[1, pl.program_id(0)] == 1)
        def _():
            sel = ((pe_ref[...] == gidx) | (po_ref[...] == gidx)).astype(_bf16)
            xb = lax.dot_general(sel, fb_ref[...], (((0,), (0,)), ((), ())),
                                 preferred_element_type=_f32).astype(_bf16)
            h = jnp.dot(xb, w1_ref[0].astype(_bf16), preferred_element_type=_f32)
            h = jnp.maximum(h + b1_ref[0], 0.0).astype(_bf16)
            out = jnp.dot(h, w2_ref[0].astype(_bf16), preferred_element_type=_f32) + b2_ref[0]
            out_ref[...] = out.astype(_bf16).astype(_f32)

    grid_spec = pltpu.PrefetchScalarGridSpec(
        num_scalar_prefetch=1,
        grid=(_NT,),
        in_specs=[
            pl.BlockSpec((_B, _D), lambda i, te: (0, 0)),
            pl.BlockSpec((_B, 1), lambda i, te: (0, 0)),
            pl.BlockSpec((_B, 1), lambda i, te: (0, 0)),
            pl.BlockSpec((1, _D, _H), lambda i, te: (te[0, i], 0, 0)),
            pl.BlockSpec((1, 1, _H), lambda i, te: (te[0, i], 0, 0)),
            pl.BlockSpec((1, _H, _D), lambda i, te: (te[0, i], 0, 0)),
            pl.BlockSpec((1, 1, _D), lambda i, te: (te[0, i], 0, 0)),
        ],
        out_specs=pl.BlockSpec((_T, _D), lambda i, te: (i, 0)),
    )
    return pl.pallas_call(
        body,
        grid_spec=grid_spec,
        out_shape=jax.ShapeDtypeStruct((_PCAP, _D), _f32),
    )(tile_expert, featb, pos_e, pos_o, W1, b1.reshape(_E, 1, _H), W2,
      b2.reshape(_E, 1, _D))


def _sc_combine(outrows, pos_e, pos_o):
    NW = 32
    TPW = _B // NW

    mesh = plsc.VectorSubcoreMesh(core_axis_name="c", subcore_axis_name="s")

    @functools.partial(
        pl.kernel,
        out_type=[jax.ShapeDtypeStruct((_B, _D), _f32),
                  jax.ShapeDtypeStruct((_B, _D), _f32)],
        mesh=mesh,
        scratch_types=[
            pltpu.VMEM((TPW,), jnp.int32),
            pltpu.VMEM((TPW, _D), _f32),
            pltpu.SemaphoreType.DMA,
        ],
    )
    def k(rows_hbm, pe_hbm, po_hbm, outa_hbm, outb_hbm, idx_v, rows_v, sem):
        wid = lax.axis_index("s") * 2 + lax.axis_index("c")
        base = wid * TPW
        pltpu.sync_copy(pe_hbm.at[pl.ds(base, TPW)], idx_v)
        pltpu.async_copy(rows_hbm.at[idx_v], rows_v, sem).wait()
        pltpu.sync_copy(rows_v, outa_hbm.at[pl.ds(base, TPW)])
        pltpu.sync_copy(po_hbm.at[pl.ds(base, TPW)], idx_v)
        pltpu.async_copy(rows_hbm.at[idx_v], rows_v, sem).wait()
        pltpu.sync_copy(rows_v, outb_hbm.at[pl.ds(base, TPW)])

    return k(outrows, pos_e, pos_o)


def _classifier(moeA, moeB, rp, Wc, bc):
    def body(ma_ref, mb_ref, rp_ref, wc_ref, bc_ref, out_ref):
        w1 = rp_ref[:, 2:3].astype(_bf16).astype(_f32)
        w2 = rp_ref[:, 3:4].astype(_bf16).astype(_f32)
        m = (w1 * ma_ref[...] + w2 * mb_ref[...]).astype(_bf16)
        out_ref[...] = jnp.dot(m, wc_ref[...].astype(_bf16),
                               preferred_element_type=_f32) + bc_ref[...]

    return pl.pallas_call(
        body,
        grid=(_NBT,),
        in_specs=[
            pl.BlockSpec((_BT, _D), lambda i: (i, 0)),
            pl.BlockSpec((_BT, _D), lambda i: (i, 0)),
            pl.BlockSpec((_BT, _LN), lambda i: (i, 0)),
            pl.BlockSpec((_D, _C), lambda i: (0, 0)),
            pl.BlockSpec((_C,), lambda i: (0,)),
        ],
        out_specs=pl.BlockSpec((_BT, _C), lambda i: (i, 0)),
        out_shape=jax.ShapeDtypeStruct((_B, _C), _f32),
    )(moeA, moeB, rp, Wc, bc)


def kernel(x, Wt, bt, Wg, bg, W1, b1, W2, b2, Wc, bc):
    Wg_pad = jnp.zeros((_D, _LN), _f32).at[:, :_E].set(Wg)
    bg_pad = jnp.zeros((_LN,), _f32).at[:_E].set(bg)

    featb, p, rp = _trunk_gate(x, Wt, bt, Wg_pad, bg_pad)
    pos_e, pos_o, tile_expert = _route_meta(rp)
    outrows = _mlp(featb, pos_e, pos_o, W1, b1, W2, b2, tile_expert)
    moeA, moeB = _sc_combine(outrows, pos_e.reshape(_B), pos_o.reshape(_B))
    logits = _classifier(moeA, moeB, rp, Wc, bc)
    return (logits, p)


# speedup vs baseline: 1.0804x; 1.0804x over previous
import functools

import jax
import jax.numpy as jnp
from jax import lax
from jax.experimental import pallas as pl
from jax.experimental.pallas import tpu as pltpu
from jax.experimental.pallas import tpu_sc as plsc

_B, _DIN, _D, _E, _H, _C = 2048, 2048, 1024, 8, 2048, 1000
_T = 256
_NT = 24
_PCAP = _T * _NT
_BT = 256
_NBT = _B // _BT
_LN = 128

_bf16 = jnp.bfloat16
_f32 = jnp.float32


def _trunk_gate(x, Wt, bt, Wg_pad, bg_pad):
    def body(x_ref, wt_ref, bt_ref, wg_ref, bg_ref, f_ref, p_ref, rp_ref):
        xb = x_ref[...].astype(_bf16)
        feat = jnp.dot(xb, wt_ref[...].astype(_bf16), preferred_element_type=_f32)
        feat = jnp.maximum(feat + bt_ref[...], 0.0)
        f_ref[...] = feat
        gl = jnp.dot(feat.astype(_bf16), wg_ref[...].astype(_bf16),
                     preferred_element_type=_f32) + bg_ref[...]
        lane = lax.broadcasted_iota(jnp.int32, (_BT, _LN), 1)
        gl = jnp.where(lane < _E, gl, -jnp.inf)
        m = jnp.max(gl, axis=1, keepdims=True)
        ex = jnp.exp(gl - m)
        p = ex / jnp.sum(ex, axis=1, keepdims=True)
        p_ref[...] = p[:, :_E]
        m1 = jnp.max(p, axis=1, keepdims=True)
        e1 = jnp.min(jnp.where(p >= m1, lane, _LN), axis=1, keepdims=True)
        p2 = jnp.where(lane == e1, -1.0, p)
        m2 = jnp.max(p2, axis=1, keepdims=True)
        e2 = jnp.min(jnp.where(p2 >= m2, lane, _LN), axis=1, keepdims=True)
        s = m1 + m2
        rp_ref[...] = (jnp.where(lane == 0, e1.astype(_f32), 0.0)
                       + jnp.where(lane == 1, e2.astype(_f32), 0.0)
                       + jnp.where(lane == 2, m1 / s, 0.0)
                       + jnp.where(lane == 3, m2 / s, 0.0))

    return pl.pallas_call(
        body,
        grid=(_NBT,),
        in_specs=[
            pl.BlockSpec((_BT, _DIN), lambda i: (i, 0)),
            pl.BlockSpec((_DIN, _D), lambda i: (0, 0)),
            pl.BlockSpec((_D,), lambda i: (0,)),
            pl.BlockSpec((_D, _LN), lambda i: (0, 0)),
            pl.BlockSpec((_LN,), lambda i: (0,)),
        ],
        out_specs=[
            pl.BlockSpec((_BT, _D), lambda i: (i, 0)),
            pl.BlockSpec((_BT, _E), lambda i: (i, 0)),
            pl.BlockSpec((_BT, _LN), lambda i: (i, 0)),
        ],
        out_shape=[
            jax.ShapeDtypeStruct((_B, _D), _f32),
            jax.ShapeDtypeStruct((_B, _E), _f32),
            jax.ShapeDtypeStruct((_B, _LN), _f32),
        ],
    )(x, Wt, bt, Wg_pad, bg_pad)


def _cumsum_axis0(x, n):
    iot = lax.broadcasted_iota(jnp.int32, x.shape, 0)
    k = 1
    while k < n:
        sh = pltpu.roll(x, k, 0)
        x = x + jnp.where(iot >= k, sh, 0.0)
        k *= 2
    return x


def _route_meta(rp):
    def body(rp_ref, pe_ref, po_ref, te_ref):
        rpv = rp_ref[...]
        lane = lax.broadcasted_iota(jnp.int32, (_B, _LN), 1)
        e1 = rpv[:, 0:1].astype(jnp.int32)
        e2 = rpv[:, 1:2].astype(jnp.int32)
        oh1 = (lane == e1).astype(_f32)
        oh2 = (lane == e2).astype(_f32)
        cum1 = _cumsum_axis0(oh1, _B)
        cum2 = _cumsum_axis0(oh2, _B)
        counts1 = cum1[_B - 1:_B, :]
        counts = counts1 + cum2[_B - 1:_B, :]
        padded = jnp.floor((counts + (_T - 1)) * (1.0 / _T)) * _T
        liot = lax.broadcasted_iota(jnp.int32, (1, _LN), 1)
        ends = padded
        k = 1
        while k < _E:
            ends = ends + jnp.where(liot >= k, pltpu.roll(ends, k, 1), 0.0)
            k *= 2
        offs = ends - padded
        pos_e = jnp.sum(oh1 * (offs + cum1), axis=1, keepdims=True) - 1.0
        pos_o = jnp.sum(oh2 * (offs + counts1 + cum2), axis=1, keepdims=True) - 1.0
        pe_ref[...] = pos_e.astype(jnp.int32)
        po_ref[...] = pos_o.astype(jnp.int32)
        tile_start = (liot * _T).astype(_f32)
        te = jnp.zeros((1, _LN), _f32)
        for e in range(_E):
            end_e = jnp.sum(jnp.where(liot == e, ends, 0.0), axis=1, keepdims=True)
            te = te + (tile_start >= end_e).astype(_f32)
        total = jnp.sum(jnp.where(liot == _E - 1, ends, 0.0), axis=1, keepdims=True)
        active = (tile_start < total).astype(jnp.int32)
        te_ref[0:1, :] = jnp.minimum(te, _E - 1).astype(jnp.int32)
        te_ref[1:2, :] = active

    return pl.pallas_call(
        body,
        grid=(1,),
        in_specs=[pl.BlockSpec((_B, _LN), lambda i: (0, 0))],
        out_specs=[
            pl.BlockSpec((_B, 1), lambda i: (0, 0)),
            pl.BlockSpec((_B, 1), lambda i: (0, 0)),
            pl.BlockSpec((2, _LN), lambda i: (0, 0)),
        ],
        out_shape=[
            jax.ShapeDtypeStruct((_B, 1), jnp.int32),
            jax.ShapeDtypeStruct((_B, 1), jnp.int32),
            jax.ShapeDtypeStruct((2, _LN), jnp.int32),
        ],
    )(rp)


def _sc_scatter_rows(feats, pos_e, pos_o):
    NW = 32
    TPW = _B // NW

    mesh = plsc.VectorSubcoreMesh(core_axis_name="c", subcore_axis_name="s")

    @functools.partial(
        pl.kernel,
        out_type=jax.ShapeDtypeStruct((_PCAP, _D), _f32),
        mesh=mesh,
        scratch_types=[
            pltpu.VMEM((TPW,), jnp.int32),
            pltpu.VMEM((TPW,), jnp.int32),
            pltpu.VMEM((TPW, _D), _f32),
            pltpu.SemaphoreType.DMA,
            pltpu.SemaphoreType.DMA,
        ],
    )
    def k(f_hbm, pe_hbm, po_hbm, out_hbm, idxa_v, idxb_v, rows_v, sema, semb):
        wid = lax.axis_index("s") * 2 + lax.axis_index("c")
        base = wid * TPW
        pltpu.sync_copy(f_hbm.at[pl.ds(base, TPW)], rows_v)
        pltpu.sync_copy(pe_hbm.at[pl.ds(base, TPW)], idxa_v)
        pltpu.sync_copy(po_hbm.at[pl.ds(base, TPW)], idxb_v)
        ca = pltpu.async_copy(rows_v, out_hbm.at[idxa_v], sema)
        cb = pltpu.async_copy(rows_v, out_hbm.at[idxb_v], semb)
        ca.wait()
        cb.wait()

    return k(feats, pos_e, pos_o)


def _mlp(xg, W1, b1, W2, b2, tile_expert):
    def body(te_ref, xg_ref, w1_ref, b1_ref, w2_ref, b2_ref, out_ref):
        @pl.when(te_ref[1, pl.program_id(0)] == 1)
        def _():
            xb = xg_ref[...].astype(_bf16)
            h = jnp.dot(xb, w1_ref[0].astype(_bf16), preferred_element_type=_f32)
            h = jnp.maximum(h + b1_ref[0], 0.0).astype(_bf16)
            out = jnp.dot(h, w2_ref[0].astype(_bf16), preferred_element_type=_f32) + b2_ref[0]
            out_ref[...] = out.astype(_bf16).astype(_f32)

    grid_spec = pltpu.PrefetchScalarGridSpec(
        num_scalar_prefetch=1,
        grid=(_NT,),
        in_specs=[
            pl.BlockSpec((_T, _D), lambda i, te: (i, 0)),
            pl.BlockSpec((1, _D, _H), lambda i, te: (te[0, i], 0, 0)),
            pl.BlockSpec((1, 1, _H), lambda i, te: (te[0, i], 0, 0)),
            pl.BlockSpec((1, _H, _D), lambda i, te: (te[0, i], 0, 0)),
            pl.BlockSpec((1, 1, _D), lambda i, te: (te[0, i], 0, 0)),
        ],
        out_specs=pl.BlockSpec((_T, _D), lambda i, te: (i, 0)),
    )
    return pl.pallas_call(
        body,
        grid_spec=grid_spec,
        out_shape=jax.ShapeDtypeStruct((_PCAP, _D), _f32),
    )(tile_expert, xg, W1, b1.reshape(_E, 1, _H), W2, b2.reshape(_E, 1, _D))


def _sc_combine(outrows, pos_e, pos_o):
    NW = 32
    TPW = _B // NW

    mesh = plsc.VectorSubcoreMesh(core_axis_name="c", subcore_axis_name="s")

    @functools.partial(
        pl.kernel,
        out_type=[jax.ShapeDtypeStruct((_B, _D), _f32),
                  jax.ShapeDtypeStruct((_B, _D), _f32)],
        mesh=mesh,
        scratch_types=[
            pltpu.VMEM((TPW,), jnp.int32),
            pltpu.VMEM((TPW, _D), _f32),
            pltpu.SemaphoreType.DMA,
        ],
    )
    def k(rows_hbm, pe_hbm, po_hbm, outa_hbm, outb_hbm, idx_v, rows_v, sem):
        wid = lax.axis_index("s") * 2 + lax.axis_index("c")
        base = wid * TPW
        pltpu.sync_copy(pe_hbm.at[pl.ds(base, TPW)], idx_v)
        pltpu.async_copy(rows_hbm.at[idx_v], rows_v, sem).wait()
        pltpu.sync_copy(rows_v, outa_hbm.at[pl.ds(base, TPW)])
        pltpu.sync_copy(po_hbm.at[pl.ds(base, TPW)], idx_v)
        pltpu.async_copy(rows_hbm.at[idx_v], rows_v, sem).wait()
        pltpu.sync_copy(rows_v, outb_hbm.at[pl.ds(base, TPW)])

    return k(outrows, pos_e, pos_o)


def _classifier(moeA, moeB, rp, Wc, bc):
    def body(ma_ref, mb_ref, rp_ref, wc_ref, bc_ref, out_ref):
        w1 = rp_ref[:, 2:3].astype(_bf16).astype(_f32)
        w2 = rp_ref[:, 3:4].astype(_bf16).astype(_f32)
        m = (w1 * ma_ref[...] + w2 * mb_ref[...]).astype(_bf16)
        out_ref[...] = jnp.dot(m, wc_ref[...].astype(_bf16),
                               preferred_element_type=_f32) + bc_ref[...]

    return pl.pallas_call(
        body,
        grid=(_NBT,),
        in_specs=[
            pl.BlockSpec((_BT, _D), lambda i: (i, 0)),
            pl.BlockSpec((_BT, _D), lambda i: (i, 0)),
            pl.BlockSpec((_BT, _LN), lambda i: (i, 0)),
            pl.BlockSpec((_D, _C), lambda i: (0, 0)),
            pl.BlockSpec((_C,), lambda i: (0,)),
        ],
        out_specs=pl.BlockSpec((_BT, _C), lambda i: (i, 0)),
        out_shape=jax.ShapeDtypeStruct((_B, _C), _f32),
    )(moeA, moeB, rp, Wc, bc)


def kernel(x, Wt, bt, Wg, bg, W1, b1, W2, b2, Wc, bc):
    Wg_pad = jnp.zeros((_D, _LN), _f32).at[:, :_E].set(Wg)
    bg_pad = jnp.zeros((_LN,), _f32).at[:_E].set(bg)

    feats, p, rp = _trunk_gate(x, Wt, bt, Wg_pad, bg_pad)
    pos_e, pos_o, tile_expert = _route_meta(rp)
    pos_e = pos_e.reshape(_B)
    pos_o = pos_o.reshape(_B)
    xg = _sc_scatter_rows(feats, pos_e, pos_o)
    outrows = _mlp(xg, W1, b1, W2, b2, tile_expert)
    moeA, moeB = _sc_combine(outrows, pos_e, pos_o)
    logits = _classifier(moeA, moeB, rp, Wc, bc)
    return (logits, p)
